# baseline (speedup 1.0000x reference)
import jax
import jax.numpy as jnp
from jax.experimental import pallas as pl
from jax.experimental.pallas import tpu as pltpu

_B = 4096
_E = 1000
_BR = 2048
_MARGIN = 0.9


def _loss_body(out_ref, tgt_ref, acc_ref, vacc_ref):
    i = pl.program_id(0)

    @pl.when(i == 0)
    def _():
        vacc_ref[...] = jnp.zeros((1, _E), jnp.float32)

    out = out_ref[...]
    tgt = tgt_ref[...]
    cls = jax.lax.broadcasted_iota(jnp.int32, (_BR, _E), 1)
    masked = jnp.where(cls == tgt, out, 0.0)
    ones_col = jnp.ones((_E, 1), jnp.float32)
    at = jax.lax.dot_general(masked, ones_col, (((1,), (0,)), ((), ())),
                             preferred_element_type=jnp.float32)
    d = jnp.maximum(out + (_MARGIN - at), 0.0)
    dd = d * d
    ones_row = jnp.ones((_BR, 1), jnp.float32)
    colsum = jax.lax.dot_general(ones_row, dd, (((0,), (0,)), ((), ())),
                                 preferred_element_type=jnp.float32)
    vacc_ref[...] += colsum

    @pl.when(i == pl.num_programs(0) - 1)
    def _():
        acc_ref[...] = jnp.full((1, 1), jnp.sum(vacc_ref[...]) * (1.0 / _B),
                                jnp.float32)


def kernel(output, target):
    tgt2d = target.reshape(_B, 1).astype(jnp.int32)
    acc = pl.pallas_call(
        _loss_body,
        grid=(_B // _BR,),
        in_specs=[
            pl.BlockSpec((_BR, _E), lambda i: (i, 0)),
            pl.BlockSpec((_BR, 1), lambda i: (i, 0)),
        ],
        out_specs=pl.BlockSpec((1, 1), lambda i: (0, 0)),
        out_shape=jax.ShapeDtypeStruct((1, 1), jnp.float32),
        scratch_shapes=[pltpu.VMEM((1, _E), jnp.float32)],
    )(output, tgt2d)
    return acc[0, 0]


# speedup vs baseline: 1.0217x; 1.0217x over previous
import jax
import jax.numpy as jnp
from jax.experimental import pallas as pl
from jax.experimental.pallas import tpu as pltpu

_B = 4096
_E = 1000
_BR = 1024
_MARGIN = 0.9


def _loss_body(out_ref, tgt_ref, acc_ref, vacc_ref):
    i = pl.program_id(0)

    @pl.when(i == 0)
    def _():
        vacc_ref[...] = jnp.zeros((8, _E), jnp.float32)

    out = out_ref[...]
    tgt = tgt_ref[...]
    cls = jax.lax.broadcasted_iota(jnp.int32, (_BR, _E), 1)
    at = jnp.sum(jnp.where(cls == tgt, out, 0.0), axis=1, keepdims=True)
    d = jnp.maximum(_MARGIN - at + out, 0.0)
    vacc_ref[...] += jnp.sum((d * d).reshape(_BR // 8, 8, _E), axis=0)

    @pl.when(i == pl.num_programs(0) - 1)
    def _():
        acc_ref[...] = jnp.full((1, 1), jnp.sum(vacc_ref[...]) * (1.0 / _B),
                                jnp.float32)


def kernel(output, target):
    tgt2d = target.reshape(_B, 1).astype(jnp.int32)
    acc = pl.pallas_call(
        _loss_body,
        grid=(_B // _BR,),
        in_specs=[
            pl.BlockSpec((_BR, _E), lambda i: (i, 0)),
            pl.BlockSpec((_BR, 1), lambda i: (i, 0)),
        ],
        out_specs=pl.BlockSpec((1, 1), lambda i: (0, 0)),
        out_shape=jax.ShapeDtypeStruct((1, 1), jnp.float32),
        scratch_shapes=[pltpu.VMEM((8, _E), jnp.float32)],
    )(output, tgt2d)
    return acc[0, 0]


# speedup vs baseline: 2.7812x; 2.7223x over previous
import jax
import jax.numpy as jnp
from jax.experimental import pallas as pl
from jax.experimental.pallas import tpu as pltpu

_B = 4096
_E = 1000
_BL = 512
_MARGIN = 0.9


def _loss_body(out_ref, tgt_ref, acc_ref, vacc_ref):
    i = pl.program_id(0)

    @pl.when(i == 0)
    def _():
        vacc_ref[...] = jnp.zeros((8, _BL), jnp.float32)

    out = out_ref[...]
    tgt = tgt_ref[...].reshape(1, _BL)
    cls = jax.lax.broadcasted_iota(jnp.int32, (_E, _BL), 0)
    at = jnp.sum(jnp.where(cls == tgt, out, 0.0), axis=0, keepdims=True)
    d = jnp.maximum((_MARGIN - at) + out, 0.0)
    vacc_ref[...] += jnp.sum((d * d).reshape(_E // 8, 8, _BL), axis=0)

    @pl.when(i == pl.num_programs(0) - 1)
    def _():
        acc_ref[...] = jnp.full((1, 1), jnp.sum(vacc_ref[...]) * (1.0 / _B),
                                jnp.float32)


def kernel(output, target):
    out_t = output.T
    acc = pl.pallas_call(
        _loss_body,
        grid=(_B // _BL,),
        in_specs=[
            pl.BlockSpec((_E, _BL), lambda i: (0, i)),
            pl.BlockSpec((_BL,), lambda i: (i,)),
        ],
        out_specs=pl.BlockSpec((1, 1), lambda i: (0, 0)),
        out_shape=jax.ShapeDtypeStruct((1, 1), jnp.float32),
        scratch_shapes=[pltpu.VMEM((8, _BL), jnp.float32)],
    )(out_t, target.astype(jnp.int32))
    return acc[0, 0]


# speedup vs baseline: 3.3854x; 1.2172x over previous
import jax
import jax.numpy as jnp
from jax.experimental import pallas as pl
from jax.experimental.pallas import tpu as pltpu

_B = 4096
_E = 1000
_BL = 1024
_MARGIN = 0.9


def _loss_body(out_ref, tgt_ref, acc_ref, vacc_ref):
    i = pl.program_id(0)

    @pl.when(i == 0)
    def _():
        vacc_ref[...] = jnp.zeros((8, _BL), jnp.float32)

    out = out_ref[...]
    tgt = tgt_ref[...].reshape(1, _BL)
    cls = jax.lax.broadcasted_iota(jnp.int32, (_E, _BL), 0)
    at = jnp.sum(jnp.where(cls == tgt, out, 0.0), axis=0, keepdims=True)
    d = jnp.maximum((_MARGIN - at) + out, 0.0)
    vacc_ref[...] += jnp.sum((d * d).reshape(_E // 8, 8, _BL), axis=0)

    @pl.when(i == pl.num_programs(0) - 1)
    def _():
        acc_ref[...] = jnp.full((1, 1), jnp.sum(vacc_ref[...]) * (1.0 / _B),
                                jnp.float32)


def kernel(output, target):
    out_t = output.T
    acc = pl.pallas_call(
        _loss_body,
        grid=(_B // _BL,),
        in_specs=[
            pl.BlockSpec((_E, _BL), lambda i: (0, i)),
            pl.BlockSpec((_BL,), lambda i: (i,)),
        ],
        out_specs=pl.BlockSpec((1, 1), lambda i: (0, 0)),
        out_shape=jax.ShapeDtypeStruct((1, 1), jnp.float32),
        scratch_shapes=[pltpu.VMEM((8, _BL), jnp.float32)],
    )(out_t, target.astype(jnp.int32))
    return acc[0, 0]
